# baseline (speedup 1.0000x reference)
import functools

import jax
import jax.numpy as jnp
from jax import lax
from jax.experimental import pallas as pl
from jax.experimental.pallas import tpu as pltpu
from jax.experimental.pallas import tpu_sc as plsc

_N = 10000
_E = 320000
_D = 128
_H = 128
_OUT = 64
_B = 1024

_NC = 2
_NS = 16
_NW = _NC * _NS
_EPW = _E // _NW
_K = 80
_NCHUNK = _EPW // _K
_NPAD = 10112
_RPT = _NPAD // _NS

_KW = 112
_EPAD = _EPW + 176
_DUMPI = _EPW + 160
_WPAD = 1280
_WRPT = _WPAD // _NS

_ROWS_TC = 1000
_GRID_TC = _N // _ROWS_TC

_sc_mesh = plsc.VectorSubcoreMesh(core_axis_name="c", subcore_axis_name="s")


_RING = 5


@functools.partial(
    pl.kernel,
    out_type=(
        jax.ShapeDtypeStruct((_NC * _NPAD,), jnp.float32),
        jax.ShapeDtypeStruct((_NW * _EPAD,), jnp.int32),
        jax.ShapeDtypeStruct((_NW * _EPAD,), jnp.int32),
        jax.ShapeDtypeStruct((_NW * 16,), jnp.int32),
    ),
    mesh=_sc_mesh,
    compiler_params=pltpu.CompilerParams(needs_layout_passes=False),
    scratch_types=[
        pltpu.VMEM((_EPW,), jnp.int32),
        pltpu.VMEM((_EPW,), jnp.int32),
        pltpu.VMEM((16,), jnp.int32),
        pltpu.VMEM((_EPAD,), jnp.int32),
        pltpu.VMEM((_EPAD,), jnp.int32),
        pltpu.VMEM((16,), jnp.int32),
        pltpu.VMEM((_K,), jnp.float32),
        pltpu.VMEM((_RPT,), jnp.float32),
        pltpu.VMEM_SHARED((_NPAD,), jnp.float32),
        [pltpu.SemaphoreType.DMA] * _RING,
    ],
)
def _deg_kernel(src_hbm, dst_hbm, win_hbm,
                deg_hbm, fsrc_hbm, fdst_hbm, fcnt_hbm,
                src_v, dst_v, win_v, fsrc_v, fdst_v, cnt_v,
                ones_v, stage_v, acc_sh, sems):
    c = lax.axis_index("c")
    s = lax.axis_index("s")
    w = c * _NS + s
    base = w * _EPW
    rstart = s * _RPT
    one = jnp.ones((16,), jnp.float32)
    zero = jnp.zeros((16,), jnp.float32)
    for i in range(_K // 16):
        ones_v[pl.ds(i * 16, 16)] = one
    for i in range(_RPT // 16):
        stage_v[pl.ds(i * 16, 16)] = zero
    stage_v[pl.ds(_RPT - 16, 16)] = zero
    pltpu.sync_copy(src_hbm.at[pl.ds(base, _EPW)], src_v)
    pltpu.sync_copy(dst_hbm.at[pl.ds(base, _EPW)], dst_v)
    pltpu.sync_copy(win_hbm, win_v)
    pltpu.sync_copy(stage_v, acc_sh.at[pl.ds(rstart, _RPT)])
    plsc.subcore_barrier()

    def fire(j, b):
        pltpu.async_copy(ones_v, acc_sh.at[dst_v.at[pl.ds(j * _K, _K)]],
                         sems[b], add=True)

    def drain(b):
        pltpu.make_async_copy(ones_v,
                              acc_sh.at[dst_v.at[pl.ds(0, _K)]],
                              sems[b]).wait()

    for b in range(_RING):
        fire(b, b)

    def round_(i, carry):
        for b in range(_RING):
            drain(b)
            fire(_RING * (i + 1) + b, b)
        return carry

    lax.fori_loop(0, _NCHUNK // _RING - 1, round_, 0)

    wl = win_v[...]
    iota16 = lax.iota(jnp.int32, 16)
    dumpv = _DUMPI + iota16
    dump = jnp.full((16,), _B, jnp.int32)
    zi = jnp.zeros((16,), jnp.int32)

    def prefill(i, carry):
        fdst_v[pl.ds(i * 16, 16)] = dump
        fsrc_v[pl.ds(i * 16, 16)] = zi
        return carry

    lax.fori_loop(0, _EPAD // 16, prefill, 0)

    def filt(i, cntv):
        dv = dst_v[pl.ds(i * 16, 16)]
        sv = src_v[pl.ds(i * 16, 16)]
        dr = dv - wl
        m = (dr >= 0) & (dr < _B)
        csum = plsc.cumsum(jnp.where(m, jnp.int32(1), jnp.int32(0)))
        pos = jnp.where(m, cntv + csum - 1, dumpv)
        plsc.store_scatter(fdst_v, [pos], dr)
        plsc.store_scatter(fsrc_v, [pos], sv)
        return cntv + plsc.all_reduce_population_count(m)

    cntv = lax.fori_loop(0, _EPW // 16, filt, jnp.zeros((16,), jnp.int32))
    cnt_v[...] = cntv
    pltpu.sync_copy(fsrc_v, fsrc_hbm.at[pl.ds(w * _EPAD, _EPAD)])
    pltpu.sync_copy(fdst_v, fdst_hbm.at[pl.ds(w * _EPAD, _EPAD)])
    pltpu.sync_copy(cnt_v, fcnt_hbm.at[pl.ds(w * 16, 16)])

    for b in range(_RING):
        drain(b)
    plsc.subcore_barrier()
    pltpu.sync_copy(acc_sh.at[pl.ds(rstart, _RPT)], stage_v)
    pltpu.sync_copy(stage_v, deg_hbm.at[pl.ds(c * _NPAD + rstart, _RPT)])


@functools.partial(
    pl.kernel,
    out_type=jax.ShapeDtypeStruct((_NC, _NPAD, _H), jnp.float32),
    mesh=_sc_mesh,
    scratch_types=[
        pltpu.VMEM((_EPW,), jnp.int32),
        pltpu.VMEM((_EPW,), jnp.int32),
        pltpu.VMEM((_K, _H), jnp.float32),
        pltpu.VMEM((_K, _H), jnp.float32),
        pltpu.VMEM_SHARED((_NPAD, _H), jnp.float32),
        pltpu.SemaphoreType.DMA,
        pltpu.SemaphoreType.DMA,
        pltpu.SemaphoreType.DMA,
        pltpu.SemaphoreType.DMA,
    ],
)
def _agg_kernel(src_hbm, dst_hbm, table_hbm, zeros_hbm, out_hbm,
                src_v, dst_v, r0, r1, acc_sh, g0, g1, ss0, ss1):
    c = lax.axis_index("c")
    s = lax.axis_index("s")
    base = (c * _NS + s) * _EPW
    rstart = s * _RPT
    pltpu.sync_copy(src_hbm.at[pl.ds(base, _EPW)], src_v)
    pltpu.sync_copy(dst_hbm.at[pl.ds(base, _EPW)], dst_v)
    pltpu.sync_copy(zeros_hbm, acc_sh.at[pl.ds(rstart, _RPT)])
    plsc.subcore_barrier()

    def gather(j, r, sem):
        pltpu.async_copy(table_hbm.at[src_v.at[pl.ds(j * _K, _K)]], r, sem)

    def drain_gather(r, sem):
        pltpu.make_async_copy(table_hbm.at[src_v.at[pl.ds(0, _K)]], r,
                              sem).wait()

    def scatter(j, r, sem):
        pltpu.async_copy(r, acc_sh.at[dst_v.at[pl.ds(j * _K, _K)]],
                         sem, add=True)

    def drain_scatter(r, sem):
        pltpu.make_async_copy(r, acc_sh.at[dst_v.at[pl.ds(0, _K)]],
                              sem).wait()

    gather(0, r0, g0)

    def chunk(i, carry):
        def even(_):
            @pl.when(i > 0)
            def _():
                drain_scatter(r1, ss1)
            gather(i + 1, r1, g1)
            drain_gather(r0, g0)
            scatter(i, r0, ss0)
            return 0

        def odd(_):
            drain_scatter(r0, ss0)
            gather(i + 1, r0, g0)
            drain_gather(r1, g1)
            scatter(i, r1, ss1)
            return 0

        return lax.cond(i % 2 == 0, even, odd, 0)

    lax.fori_loop(0, _NCHUNK - 1, chunk, 0)
    drain_scatter(r1, ss1)
    drain_gather(r0, g0)
    scatter(_NCHUNK - 1, r0, ss0)
    drain_scatter(r0, ss0)
    plsc.subcore_barrier()
    pltpu.sync_copy(acc_sh.at[pl.ds(rstart, _RPT)],
                    out_hbm.at[c, pl.ds(rstart, _RPT)])


@functools.partial(
    pl.kernel,
    out_type=jax.ShapeDtypeStruct((_NC, _WPAD, _H), jnp.float32),
    mesh=_sc_mesh,
    compiler_params=pltpu.CompilerParams(needs_layout_passes=False),
    scratch_types=[
        pltpu.VMEM((_EPAD,), jnp.int32),
        pltpu.VMEM((_EPAD,), jnp.int32),
        pltpu.VMEM((16,), jnp.int32),
        pltpu.VMEM((_KW, _H), jnp.float32),
        pltpu.VMEM((_KW, _H), jnp.float32),
        pltpu.VMEM_SHARED((_WPAD, _H), jnp.float32),
        pltpu.SemaphoreType.DMA,
        pltpu.SemaphoreType.DMA,
        pltpu.SemaphoreType.DMA,
        pltpu.SemaphoreType.DMA,
    ],
)
def _wagg_kernel(fsrc_hbm, fdst_hbm, fcnt_hbm, table_hbm, zeros_hbm, out_hbm,
                 src_v, dst_v, cnt_v, r0, r1, acc_sh, g0, g1, ss0, ss1):
    c = lax.axis_index("c")
    s = lax.axis_index("s")
    w = c * _NS + s
    pltpu.sync_copy(fsrc_hbm.at[pl.ds(w * _EPAD, _EPAD)], src_v)
    pltpu.sync_copy(fdst_hbm.at[pl.ds(w * _EPAD, _EPAD)], dst_v)
    pltpu.sync_copy(fcnt_hbm.at[pl.ds(w * 16, 16)], cnt_v)
    pltpu.sync_copy(zeros_hbm.at[pl.ds(0, _WRPT)],
                    acc_sh.at[pl.ds(s * _WRPT, _WRPT)])
    plsc.subcore_barrier()

    cnt = jnp.max(cnt_v[...])
    last = jnp.maximum((cnt + (_KW - 1)) // _KW, 1) - 1

    def gather(j, r, sem):
        pltpu.async_copy(table_hbm.at[src_v.at[pl.ds(j * _KW, _KW)]], r, sem)

    def drain_gather(r, sem):
        pltpu.make_async_copy(table_hbm.at[src_v.at[pl.ds(0, _K)]], r,
                              sem).wait()

    def scatter(j, r, sem):
        pltpu.async_copy(r, acc_sh.at[dst_v.at[pl.ds(j * _KW, _KW)]],
                         sem, add=True)

    def drain_scatter(r, sem):
        pltpu.make_async_copy(r, acc_sh.at[dst_v.at[pl.ds(0, _K)]],
                              sem).wait()

    gather(0, r0, g0)

    def chunk(i, carry):
        def even(_):
            @pl.when(i > 0)
            def _():
                drain_scatter(r1, ss1)
            gather(i + 1, r1, g1)
            drain_gather(r0, g0)
            scatter(i, r0, ss0)
            return 0

        def odd(_):
            drain_scatter(r0, ss0)
            gather(i + 1, r0, g0)
            drain_gather(r1, g1)
            scatter(i, r1, ss1)
            return 0

        return lax.cond(i % 2 == 0, even, odd, 0)

    lax.fori_loop(0, last, chunk, 0)

    def ep_even(_):
        @pl.when(last > 0)
        def _():
            drain_scatter(r1, ss1)
        drain_gather(r0, g0)
        scatter(last, r0, ss0)
        drain_scatter(r0, ss0)
        return 0

    def ep_odd(_):
        drain_scatter(r0, ss0)
        drain_gather(r1, g1)
        scatter(last, r1, ss1)
        drain_scatter(r1, ss1)
        return 0

    lax.cond(last % 2 == 0, ep_even, ep_odd, 0)
    plsc.subcore_barrier()
    pltpu.sync_copy(acc_sh.at[pl.ds(s * _WRPT, _WRPT)],
                    out_hbm.at[c, pl.ds(s * _WRPT, _WRPT)])


def _mm_body(x_ref, w_ref, o_ref):
    o_ref[...] = jnp.dot(x_ref[...], w_ref[...],
                         preferred_element_type=jnp.float32)


_mm_call = pl.pallas_call(
    _mm_body,
    grid=(_GRID_TC,),
    in_specs=[
        pl.BlockSpec((_ROWS_TC, _D), lambda i: (i, 0)),
        pl.BlockSpec((_D, _H), lambda i: (0, 0)),
    ],
    out_specs=pl.BlockSpec((_ROWS_TC, _H), lambda i: (i, 0)),
    out_shape=jax.ShapeDtypeStruct((_N, _H), jnp.float32),
)


def _dinv_of(deg_ref):
    d = deg_ref[0] + deg_ref[1] + 1.0
    return lax.rsqrt(d)


def _scale_body(deg_ref, xw_ref, o_ref):
    o_ref[...] = xw_ref[...] * _dinv_of(deg_ref)


_scale_call = pl.pallas_call(
    _scale_body,
    grid=(_GRID_TC,),
    in_specs=[
        pl.BlockSpec((_NC, _ROWS_TC, 1), lambda i: (0, i, 0)),
        pl.BlockSpec((_ROWS_TC, _H), lambda i: (i, 0)),
    ],
    out_specs=pl.BlockSpec((_ROWS_TC, _H), lambda i: (i, 0)),
    out_shape=jax.ShapeDtypeStruct((_N, _H), jnp.float32),
)


def _mid_body(deg_ref, acc_ref, xws_ref, b1_ref, w2_ref, o_ref):
    dinv = _dinv_of(deg_ref)
    h1 = acc_ref[0] + acc_ref[1] + xws_ref[...]
    h1 = jnp.maximum(h1 * dinv + b1_ref[...], 0.0)
    o_ref[...] = jnp.dot(h1, w2_ref[...],
                         preferred_element_type=jnp.float32) * dinv


_mid_call = pl.pallas_call(
    _mid_body,
    grid=(_GRID_TC,),
    in_specs=[
        pl.BlockSpec((_NC, _ROWS_TC, 1), lambda i: (0, i, 0)),
        pl.BlockSpec((_NC, _ROWS_TC, _H), lambda i: (0, i, 0)),
        pl.BlockSpec((_ROWS_TC, _H), lambda i: (i, 0)),
        pl.BlockSpec((1, _H), lambda i: (0, 0)),
        pl.BlockSpec((_H, _H), lambda i: (0, 0)),
    ],
    out_specs=pl.BlockSpec((_ROWS_TC, _H), lambda i: (i, 0)),
    out_shape=jax.ShapeDtypeStruct((_N, _H), jnp.float32),
)


def _head_body(deg_ref, acc_ref, xws_ref, b2_ref, wm1_ref, bm1_ref,
               wm2_ref, bm2_ref, o_ref):
    dinv = _dinv_of(deg_ref)
    h2 = (acc_ref[0] + acc_ref[1] + xws_ref[...]) * dinv + b2_ref[...]
    t = jnp.maximum(jnp.dot(h2, wm1_ref[...],
                            preferred_element_type=jnp.float32)
                    + bm1_ref[...], 0.0)
    o = jnp.dot(t, wm2_ref[...],
                preferred_element_type=jnp.float32) + bm2_ref[...]
    o = o - jnp.max(o, axis=1, keepdims=True)
    e = jnp.exp(o)
    o_ref[...] = e / jnp.sum(e, axis=1, keepdims=True)


_head_call = pl.pallas_call(
    _head_body,
    in_specs=[
        pl.BlockSpec((_NC, _B, 1), lambda: (0, 0, 0)),
        pl.BlockSpec((_NC, _B, _H), lambda: (0, 0, 0)),
        pl.BlockSpec((_B, _H), lambda: (0, 0)),
        pl.BlockSpec((1, _H), lambda: (0, 0)),
        pl.BlockSpec((_H, _H), lambda: (0, 0)),
        pl.BlockSpec((1, _H), lambda: (0, 0)),
        pl.BlockSpec((_H, _OUT), lambda: (0, 0)),
        pl.BlockSpec((1, _OUT), lambda: (0, 0)),
    ],
    out_specs=pl.BlockSpec((_B, _OUT), lambda: (0, 0)),
    out_shape=jax.ShapeDtypeStruct((_B, _OUT), jnp.float32),
)


def kernel(x, edge_index, n2v, batch_size,
           W1, b1, W2, b2, Wm1, bm1, Wm2, bm2):
    del n2v
    e = edge_index.astype(jnp.int32)
    src, dst = e[0], e[1]
    zerosH = jnp.zeros((_RPT, _H), jnp.float32)

    wl = jnp.clip(batch_size - _B, 0, _N - _B).astype(jnp.int32)
    win16 = jnp.broadcast_to(wl, (16,))
    deg_flat, fsrc, fdst, fcnt = _deg_kernel(src, dst, win16)
    deg3 = deg_flat.reshape(_NC, _NPAD, 1)
    xw1 = _mm_call(x, W1)
    xws1 = _scale_call(deg3, xw1)
    acc1 = _agg_kernel(src, dst, xws1, zerosH)
    xws2 = _mid_call(deg3, acc1, xws1, b1.reshape(1, _H), W2)
    acc_w = _wagg_kernel(fsrc, fdst, fcnt, xws2, zerosH)

    deg_w = lax.dynamic_slice_in_dim(deg3, wl, _B, axis=1)
    xws_w = lax.dynamic_slice_in_dim(xws2, wl, _B, axis=0)
    return _head_call(deg_w, acc_w[:, :_B], xws_w, b2.reshape(1, _H),
                      Wm1, bm1.reshape(1, _H), Wm2, bm2.reshape(1, _OUT))


# speedup vs baseline: 1.1033x; 1.1033x over previous
import functools

import jax
import jax.numpy as jnp
from jax import lax
from jax.experimental import pallas as pl
from jax.experimental.pallas import tpu as pltpu
from jax.experimental.pallas import tpu_sc as plsc

_N = 10000
_E = 320000
_D = 128
_H = 128
_OUT = 64
_B = 1024

_NC = 2
_NS = 16
_NW = _NC * _NS
_EPW = _E // _NW
_K = 80
_NCHUNK = _EPW // _K
_NPAD = 10112
_RPT = _NPAD // _NS

_KW = 80
_EPAD = _EPW + 176
_DUMPI = _EPW + 160
_WPAD = 1280
_WRPT = _WPAD // _NS

_ROWS_TC = 1000
_GRID_TC = _N // _ROWS_TC

_sc_mesh = plsc.VectorSubcoreMesh(core_axis_name="c", subcore_axis_name="s")


_RING = 5


@functools.partial(
    pl.kernel,
    out_type=(
        jax.ShapeDtypeStruct((_NC * _NPAD,), jnp.float32),
        jax.ShapeDtypeStruct((_NW * _EPAD,), jnp.int32),
        jax.ShapeDtypeStruct((_NW * _EPAD,), jnp.int32),
        jax.ShapeDtypeStruct((_NW * 16,), jnp.int32),
    ),
    mesh=_sc_mesh,
    compiler_params=pltpu.CompilerParams(needs_layout_passes=False),
    scratch_types=[
        pltpu.VMEM((_EPW,), jnp.int32),
        pltpu.VMEM((_EPW,), jnp.int32),
        pltpu.VMEM((16,), jnp.int32),
        pltpu.VMEM((_EPAD,), jnp.int32),
        pltpu.VMEM((_EPAD,), jnp.int32),
        pltpu.VMEM((16,), jnp.int32),
        pltpu.VMEM((_K,), jnp.float32),
        pltpu.VMEM((_RPT,), jnp.float32),
        pltpu.VMEM_SHARED((_NPAD,), jnp.float32),
        [pltpu.SemaphoreType.DMA] * _RING,
    ],
)
def _deg_kernel(src_hbm, dst_hbm, win_hbm,
                deg_hbm, fsrc_hbm, fdst_hbm, fcnt_hbm,
                src_v, dst_v, win_v, fsrc_v, fdst_v, cnt_v,
                ones_v, stage_v, acc_sh, sems):
    c = lax.axis_index("c")
    s = lax.axis_index("s")
    w = c * _NS + s
    base = w * _EPW
    rstart = s * _RPT
    one = jnp.ones((16,), jnp.float32)
    zero = jnp.zeros((16,), jnp.float32)
    for i in range(_K // 16):
        ones_v[pl.ds(i * 16, 16)] = one
    for i in range(_RPT // 16):
        stage_v[pl.ds(i * 16, 16)] = zero
    stage_v[pl.ds(_RPT - 16, 16)] = zero
    pltpu.sync_copy(src_hbm.at[pl.ds(base, _EPW)], src_v)
    pltpu.sync_copy(dst_hbm.at[pl.ds(base, _EPW)], dst_v)
    pltpu.sync_copy(win_hbm, win_v)
    pltpu.sync_copy(stage_v, acc_sh.at[pl.ds(rstart, _RPT)])
    plsc.subcore_barrier()

    def fire(j, b):
        pltpu.async_copy(ones_v, acc_sh.at[dst_v.at[pl.ds(j * _K, _K)]],
                         sems[b], add=True)

    def drain(b):
        pltpu.make_async_copy(ones_v,
                              acc_sh.at[dst_v.at[pl.ds(0, _K)]],
                              sems[b]).wait()

    for b in range(_RING):
        fire(b, b)

    def round_(i, carry):
        for b in range(_RING):
            drain(b)
            fire(_RING * (i + 1) + b, b)
        return carry

    lax.fori_loop(0, _NCHUNK // _RING - 1, round_, 0)

    wl = win_v[...]
    iota16 = lax.iota(jnp.int32, 16)
    dumpv = _DUMPI + iota16
    dump = jnp.full((16,), _B, jnp.int32)
    zi = jnp.zeros((16,), jnp.int32)

    def prefill(i, carry):
        fdst_v[pl.ds(i * 16, 16)] = dump
        fsrc_v[pl.ds(i * 16, 16)] = zi
        return carry

    lax.fori_loop(0, _EPAD // 16, prefill, 0)

    def filt(i, cntv):
        dv = dst_v[pl.ds(i * 16, 16)]
        sv = src_v[pl.ds(i * 16, 16)]
        dr = dv - wl
        m = (dr >= 0) & (dr < _B)
        csum = plsc.cumsum(jnp.where(m, jnp.int32(1), jnp.int32(0)))
        pos = jnp.where(m, cntv + csum - 1, dumpv)
        plsc.store_scatter(fdst_v, [pos], dr)
        plsc.store_scatter(fsrc_v, [pos], sv)
        return cntv + plsc.all_reduce_population_count(m)

    cntv = lax.fori_loop(0, _EPW // 16, filt, jnp.zeros((16,), jnp.int32))
    cnt_v[...] = cntv
    pltpu.sync_copy(fsrc_v, fsrc_hbm.at[pl.ds(w * _EPAD, _EPAD)])
    pltpu.sync_copy(fdst_v, fdst_hbm.at[pl.ds(w * _EPAD, _EPAD)])
    pltpu.sync_copy(cnt_v, fcnt_hbm.at[pl.ds(w * 16, 16)])

    for b in range(_RING):
        drain(b)
    plsc.subcore_barrier()
    pltpu.sync_copy(acc_sh.at[pl.ds(rstart, _RPT)], stage_v)
    pltpu.sync_copy(stage_v, deg_hbm.at[pl.ds(c * _NPAD + rstart, _RPT)])


@functools.partial(
    pl.kernel,
    out_type=jax.ShapeDtypeStruct((_NC, _NPAD, _H), jnp.float32),
    mesh=_sc_mesh,
    scratch_types=[
        pltpu.VMEM((_EPW,), jnp.int32),
        pltpu.VMEM((_EPW,), jnp.int32),
        pltpu.VMEM((_K, _H), jnp.float32),
        pltpu.VMEM((_K, _H), jnp.float32),
        pltpu.VMEM_SHARED((_NPAD, _H), jnp.float32),
        pltpu.SemaphoreType.DMA,
        pltpu.SemaphoreType.DMA,
        pltpu.SemaphoreType.DMA,
        pltpu.SemaphoreType.DMA,
    ],
)
def _agg_kernel(src_hbm, dst_hbm, table_hbm, zeros_hbm, out_hbm,
                src_v, dst_v, r0, r1, acc_sh, g0, g1, ss0, ss1):
    c = lax.axis_index("c")
    s = lax.axis_index("s")
    base = (c * _NS + s) * _EPW
    rstart = s * _RPT
    pltpu.sync_copy(src_hbm.at[pl.ds(base, _EPW)], src_v)
    pltpu.sync_copy(dst_hbm.at[pl.ds(base, _EPW)], dst_v)
    pltpu.sync_copy(zeros_hbm, acc_sh.at[pl.ds(rstart, _RPT)])
    plsc.subcore_barrier()

    def gather(j, r, sem):
        pltpu.async_copy(table_hbm.at[src_v.at[pl.ds(j * _K, _K)]], r, sem)

    def drain_gather(r, sem):
        pltpu.make_async_copy(table_hbm.at[src_v.at[pl.ds(0, _K)]], r,
                              sem).wait()

    def scatter(j, r, sem):
        pltpu.async_copy(r, acc_sh.at[dst_v.at[pl.ds(j * _K, _K)]],
                         sem, add=True)

    def drain_scatter(r, sem):
        pltpu.make_async_copy(r, acc_sh.at[dst_v.at[pl.ds(0, _K)]],
                              sem).wait()

    gather(0, r0, g0)

    def chunk(i, carry):
        def even(_):
            @pl.when(i > 0)
            def _():
                drain_scatter(r1, ss1)
            gather(i + 1, r1, g1)
            drain_gather(r0, g0)
            scatter(i, r0, ss0)
            return 0

        def odd(_):
            drain_scatter(r0, ss0)
            gather(i + 1, r0, g0)
            drain_gather(r1, g1)
            scatter(i, r1, ss1)
            return 0

        return lax.cond(i % 2 == 0, even, odd, 0)

    lax.fori_loop(0, _NCHUNK - 1, chunk, 0)
    drain_scatter(r1, ss1)
    drain_gather(r0, g0)
    scatter(_NCHUNK - 1, r0, ss0)
    drain_scatter(r0, ss0)
    plsc.subcore_barrier()
    pltpu.sync_copy(acc_sh.at[pl.ds(rstart, _RPT)],
                    out_hbm.at[c, pl.ds(rstart, _RPT)])


@functools.partial(
    pl.kernel,
    out_type=jax.ShapeDtypeStruct((_NC, _WPAD, _H), jnp.float32),
    mesh=_sc_mesh,
    compiler_params=pltpu.CompilerParams(needs_layout_passes=False),
    scratch_types=[
        pltpu.VMEM((_EPAD,), jnp.int32),
        pltpu.VMEM((_EPAD,), jnp.int32),
        pltpu.VMEM((16,), jnp.int32),
        pltpu.VMEM((_KW, _H), jnp.float32),
        pltpu.VMEM((_KW, _H), jnp.float32),
        pltpu.VMEM_SHARED((_WPAD, _H), jnp.float32),
        pltpu.SemaphoreType.DMA,
        pltpu.SemaphoreType.DMA,
        pltpu.SemaphoreType.DMA,
        pltpu.SemaphoreType.DMA,
    ],
)
def _wagg_kernel(fsrc_hbm, fdst_hbm, fcnt_hbm, table_hbm, zeros_hbm, out_hbm,
                 src_v, dst_v, cnt_v, r0, r1, acc_sh, g0, g1, ss0, ss1):
    c = lax.axis_index("c")
    s = lax.axis_index("s")
    w = c * _NS + s
    pltpu.sync_copy(fsrc_hbm.at[pl.ds(w * _EPAD, _EPAD)], src_v)
    pltpu.sync_copy(fdst_hbm.at[pl.ds(w * _EPAD, _EPAD)], dst_v)
    pltpu.sync_copy(fcnt_hbm.at[pl.ds(w * 16, 16)], cnt_v)
    pltpu.sync_copy(zeros_hbm.at[pl.ds(0, _WRPT)],
                    acc_sh.at[pl.ds(s * _WRPT, _WRPT)])
    plsc.subcore_barrier()

    cnt = jnp.max(cnt_v[...])
    last = jnp.maximum((cnt + (_KW - 1)) // _KW, 1) - 1

    def gather(j, r, sem):
        pltpu.async_copy(table_hbm.at[src_v.at[pl.ds(j * _KW, _KW)]], r, sem)

    def drain_gather(r, sem):
        pltpu.make_async_copy(table_hbm.at[src_v.at[pl.ds(0, _K)]], r,
                              sem).wait()

    def scatter(j, r, sem):
        pltpu.async_copy(r, acc_sh.at[dst_v.at[pl.ds(j * _KW, _KW)]],
                         sem, add=True)

    def drain_scatter(r, sem):
        pltpu.make_async_copy(r, acc_sh.at[dst_v.at[pl.ds(0, _K)]],
                              sem).wait()

    gather(0, r0, g0)

    def chunk(i, carry):
        def even(_):
            @pl.when(i > 0)
            def _():
                drain_scatter(r1, ss1)
            gather(i + 1, r1, g1)
            drain_gather(r0, g0)
            scatter(i, r0, ss0)
            return 0

        def odd(_):
            drain_scatter(r0, ss0)
            gather(i + 1, r0, g0)
            drain_gather(r1, g1)
            scatter(i, r1, ss1)
            return 0

        return lax.cond(i % 2 == 0, even, odd, 0)

    lax.fori_loop(0, last, chunk, 0)

    def ep_even(_):
        @pl.when(last > 0)
        def _():
            drain_scatter(r1, ss1)
        drain_gather(r0, g0)
        scatter(last, r0, ss0)
        drain_scatter(r0, ss0)
        return 0

    def ep_odd(_):
        drain_scatter(r0, ss0)
        drain_gather(r1, g1)
        scatter(last, r1, ss1)
        drain_scatter(r1, ss1)
        return 0

    lax.cond(last % 2 == 0, ep_even, ep_odd, 0)
    plsc.subcore_barrier()
    pltpu.sync_copy(acc_sh.at[pl.ds(s * _WRPT, _WRPT)],
                    out_hbm.at[c, pl.ds(s * _WRPT, _WRPT)])


def _mm_body(x_ref, w_ref, o_ref):
    o_ref[...] = jnp.dot(x_ref[...], w_ref[...],
                         preferred_element_type=jnp.float32)


_mm_call = pl.pallas_call(
    _mm_body,
    grid=(_GRID_TC,),
    in_specs=[
        pl.BlockSpec((_ROWS_TC, _D), lambda i: (i, 0)),
        pl.BlockSpec((_D, _H), lambda i: (0, 0)),
    ],
    out_specs=pl.BlockSpec((_ROWS_TC, _H), lambda i: (i, 0)),
    out_shape=jax.ShapeDtypeStruct((_N, _H), jnp.float32),
)


def _dinv_of(deg_ref):
    d = deg_ref[0] + deg_ref[1] + 1.0
    return lax.rsqrt(d)


def _scale_body(deg_ref, xw_ref, o_ref):
    o_ref[...] = xw_ref[...] * _dinv_of(deg_ref)


_scale_call = pl.pallas_call(
    _scale_body,
    grid=(_GRID_TC,),
    in_specs=[
        pl.BlockSpec((_NC, _ROWS_TC, 1), lambda i: (0, i, 0)),
        pl.BlockSpec((_ROWS_TC, _H), lambda i: (i, 0)),
    ],
    out_specs=pl.BlockSpec((_ROWS_TC, _H), lambda i: (i, 0)),
    out_shape=jax.ShapeDtypeStruct((_N, _H), jnp.float32),
)


def _mid_body(deg_ref, acc_ref, xws_ref, b1_ref, w2_ref, o_ref):
    dinv = _dinv_of(deg_ref)
    h1 = acc_ref[0] + acc_ref[1] + xws_ref[...]
    h1 = jnp.maximum(h1 * dinv + b1_ref[...], 0.0)
    o_ref[...] = jnp.dot(h1, w2_ref[...],
                         preferred_element_type=jnp.float32) * dinv


_mid_call = pl.pallas_call(
    _mid_body,
    grid=(_GRID_TC,),
    in_specs=[
        pl.BlockSpec((_NC, _ROWS_TC, 1), lambda i: (0, i, 0)),
        pl.BlockSpec((_NC, _ROWS_TC, _H), lambda i: (0, i, 0)),
        pl.BlockSpec((_ROWS_TC, _H), lambda i: (i, 0)),
        pl.BlockSpec((1, _H), lambda i: (0, 0)),
        pl.BlockSpec((_H, _H), lambda i: (0, 0)),
    ],
    out_specs=pl.BlockSpec((_ROWS_TC, _H), lambda i: (i, 0)),
    out_shape=jax.ShapeDtypeStruct((_N, _H), jnp.float32),
)


def _head_body(deg_ref, acc_ref, xws_ref, b2_ref, wm1_ref, bm1_ref,
               wm2_ref, bm2_ref, o_ref):
    dinv = _dinv_of(deg_ref)
    h2 = (acc_ref[0] + acc_ref[1] + xws_ref[...]) * dinv + b2_ref[...]
    t = jnp.maximum(jnp.dot(h2, wm1_ref[...],
                            preferred_element_type=jnp.float32)
                    + bm1_ref[...], 0.0)
    o = jnp.dot(t, wm2_ref[...],
                preferred_element_type=jnp.float32) + bm2_ref[...]
    o = o - jnp.max(o, axis=1, keepdims=True)
    e = jnp.exp(o)
    o_ref[...] = e / jnp.sum(e, axis=1, keepdims=True)


_head_call = pl.pallas_call(
    _head_body,
    in_specs=[
        pl.BlockSpec((_NC, _B, 1), lambda: (0, 0, 0)),
        pl.BlockSpec((_NC, _B, _H), lambda: (0, 0, 0)),
        pl.BlockSpec((_B, _H), lambda: (0, 0)),
        pl.BlockSpec((1, _H), lambda: (0, 0)),
        pl.BlockSpec((_H, _H), lambda: (0, 0)),
        pl.BlockSpec((1, _H), lambda: (0, 0)),
        pl.BlockSpec((_H, _OUT), lambda: (0, 0)),
        pl.BlockSpec((1, _OUT), lambda: (0, 0)),
    ],
    out_specs=pl.BlockSpec((_B, _OUT), lambda: (0, 0)),
    out_shape=jax.ShapeDtypeStruct((_B, _OUT), jnp.float32),
)


def kernel(x, edge_index, n2v, batch_size,
           W1, b1, W2, b2, Wm1, bm1, Wm2, bm2):
    del n2v
    e = edge_index.astype(jnp.int32)
    src, dst = e[0], e[1]
    zerosH = jnp.zeros((_RPT, _H), jnp.float32)

    wl = jnp.clip(batch_size - _B, 0, _N - _B).astype(jnp.int32)
    win16 = jnp.broadcast_to(wl, (16,))
    deg_flat, fsrc, fdst, fcnt = _deg_kernel(src, dst, win16)
    deg3 = deg_flat.reshape(_NC, _NPAD, 1)
    xw1 = _mm_call(x, W1)
    xws1 = _scale_call(deg3, xw1)
    acc1 = _agg_kernel(src, dst, xws1, zerosH)
    xws2 = _mid_call(deg3, acc1, xws1, b1.reshape(1, _H), W2)
    acc_w = _wagg_kernel(fsrc, fdst, fcnt, xws2, zerosH)

    deg_w = lax.dynamic_slice_in_dim(deg3, wl, _B, axis=1)
    xws_w = lax.dynamic_slice_in_dim(xws2, wl, _B, axis=0)
    return _head_call(deg_w, acc_w[:, :_B], xws_w, b2.reshape(1, _H),
                      Wm1, bm1.reshape(1, _H), Wm2, bm2.reshape(1, _OUT))


# speedup vs baseline: 1.1078x; 1.0041x over previous
import functools

import jax
import jax.numpy as jnp
from jax import lax
from jax.experimental import pallas as pl
from jax.experimental.pallas import tpu as pltpu
from jax.experimental.pallas import tpu_sc as plsc

_N = 10000
_E = 320000
_D = 128
_H = 128
_OUT = 64
_B = 1024

_NC = 2
_NS = 16
_NW = _NC * _NS
_EPW = _E // _NW
_K = 80
_NCHUNK = _EPW // _K
_NPAD = 10112
_RPT = _NPAD // _NS

_KW = 80
_EPAD = _EPW + 176
_DUMPI = _EPW + 160
_WPAD = 1280
_WRPT = _WPAD // _NS

_ROWS_TC = 1000
_GRID_TC = _N // _ROWS_TC

_sc_mesh = plsc.VectorSubcoreMesh(core_axis_name="c", subcore_axis_name="s")


_RING = 5


@functools.partial(
    pl.kernel,
    out_type=(
        jax.ShapeDtypeStruct((_NC * _NPAD,), jnp.float32),
        jax.ShapeDtypeStruct((_NW * _EPAD,), jnp.int32),
        jax.ShapeDtypeStruct((_NW * _EPAD,), jnp.int32),
        jax.ShapeDtypeStruct((_NW * 16,), jnp.int32),
    ),
    mesh=_sc_mesh,
    compiler_params=pltpu.CompilerParams(needs_layout_passes=False),
    scratch_types=[
        pltpu.VMEM((_EPW,), jnp.int32),
        pltpu.VMEM((_EPW,), jnp.int32),
        pltpu.VMEM((16,), jnp.int32),
        pltpu.VMEM((_EPAD,), jnp.int32),
        pltpu.VMEM((_EPAD,), jnp.int32),
        pltpu.VMEM((16,), jnp.int32),
        pltpu.VMEM((_K,), jnp.float32),
        pltpu.VMEM((_RPT,), jnp.float32),
        pltpu.VMEM_SHARED((_NPAD,), jnp.float32),
        [pltpu.SemaphoreType.DMA] * _RING,
    ],
)
def _deg_kernel(src_hbm, dst_hbm, win_hbm,
                deg_hbm, fsrc_hbm, fdst_hbm, fcnt_hbm,
                src_v, dst_v, win_v, fsrc_v, fdst_v, cnt_v,
                ones_v, stage_v, acc_sh, sems):
    c = lax.axis_index("c")
    s = lax.axis_index("s")
    w = c * _NS + s
    base = w * _EPW
    rstart = s * _RPT
    one = jnp.ones((16,), jnp.float32)
    zero = jnp.zeros((16,), jnp.float32)
    for i in range(_K // 16):
        ones_v[pl.ds(i * 16, 16)] = one
    for i in range(_RPT // 16):
        stage_v[pl.ds(i * 16, 16)] = zero
    stage_v[pl.ds(_RPT - 16, 16)] = zero
    pltpu.sync_copy(src_hbm.at[pl.ds(base, _EPW)], src_v)
    pltpu.sync_copy(dst_hbm.at[pl.ds(base, _EPW)], dst_v)
    pltpu.sync_copy(win_hbm, win_v)
    pltpu.sync_copy(stage_v, acc_sh.at[pl.ds(rstart, _RPT)])
    plsc.subcore_barrier()

    def fire(j, b):
        pltpu.async_copy(ones_v, acc_sh.at[dst_v.at[pl.ds(j * _K, _K)]],
                         sems[b], add=True)

    def drain(b):
        pltpu.make_async_copy(ones_v,
                              acc_sh.at[dst_v.at[pl.ds(0, _K)]],
                              sems[b]).wait()

    for b in range(_RING):
        fire(b, b)

    def round_(i, carry):
        for b in range(_RING):
            drain(b)
            fire(_RING * (i + 1) + b, b)
        return carry

    lax.fori_loop(0, _NCHUNK // _RING - 1, round_, 0)

    wl = win_v[...]
    iota16 = lax.iota(jnp.int32, 16)
    dumpv = _DUMPI + iota16
    dump = jnp.full((16,), _B, jnp.int32)
    zi = jnp.zeros((16,), jnp.int32)

    def prefill(i, carry):
        fdst_v[pl.ds(i * 16, 16)] = dump
        fsrc_v[pl.ds(i * 16, 16)] = zi
        return carry

    lax.fori_loop(0, _EPAD // 16, prefill, 0)

    def filt(i, cntv):
        dv = dst_v[pl.ds(i * 16, 16)]
        sv = src_v[pl.ds(i * 16, 16)]
        dr = dv - wl
        m = (dr >= 0) & (dr < _B)
        csum = plsc.cumsum(jnp.where(m, jnp.int32(1), jnp.int32(0)))
        pos = jnp.where(m, cntv + csum - 1, dumpv)
        plsc.store_scatter(fdst_v, [pos], dr)
        plsc.store_scatter(fsrc_v, [pos], sv)
        return cntv + plsc.all_reduce_population_count(m)

    cntv = lax.fori_loop(0, _EPW // 16, filt, jnp.zeros((16,), jnp.int32))
    cnt_v[...] = cntv
    pltpu.sync_copy(fsrc_v, fsrc_hbm.at[pl.ds(w * _EPAD, _EPAD)])
    pltpu.sync_copy(fdst_v, fdst_hbm.at[pl.ds(w * _EPAD, _EPAD)])
    pltpu.sync_copy(cnt_v, fcnt_hbm.at[pl.ds(w * 16, 16)])

    for b in range(_RING):
        drain(b)
    plsc.subcore_barrier()
    pltpu.sync_copy(acc_sh.at[pl.ds(rstart, _RPT)], stage_v)
    pltpu.sync_copy(stage_v, deg_hbm.at[pl.ds(c * _NPAD + rstart, _RPT)])


@functools.partial(
    pl.kernel,
    out_type=jax.ShapeDtypeStruct((_NC, _NPAD, _H), jnp.float32),
    mesh=_sc_mesh,
    scratch_types=[
        pltpu.VMEM((_EPW,), jnp.int32),
        pltpu.VMEM((_EPW,), jnp.int32),
        pltpu.VMEM((_K, _H), jnp.float32),
        pltpu.VMEM((_K, _H), jnp.float32),
        pltpu.VMEM_SHARED((_NPAD, _H), jnp.float32),
        pltpu.SemaphoreType.DMA,
        pltpu.SemaphoreType.DMA,
        pltpu.SemaphoreType.DMA,
        pltpu.SemaphoreType.DMA,
    ],
)
def _agg_kernel(src_hbm, dst_hbm, table_hbm, zeros_hbm, out_hbm,
                src_v, dst_v, r0, r1, acc_sh, g0, g1, ss0, ss1):
    c = lax.axis_index("c")
    s = lax.axis_index("s")
    base = (c * _NS + s) * _EPW
    rstart = s * _RPT
    pltpu.sync_copy(src_hbm.at[pl.ds(base, _EPW)], src_v)
    pltpu.sync_copy(dst_hbm.at[pl.ds(base, _EPW)], dst_v)
    pltpu.sync_copy(zeros_hbm, acc_sh.at[pl.ds(rstart, _RPT)])
    plsc.subcore_barrier()

    def gather(j, r, sem):
        pltpu.async_copy(table_hbm.at[src_v.at[pl.ds(j * _K, _K)]], r, sem)

    def drain_gather(r, sem):
        pltpu.make_async_copy(table_hbm.at[src_v.at[pl.ds(0, _K)]], r,
                              sem).wait()

    def scatter(j, r, sem):
        pltpu.async_copy(r, acc_sh.at[dst_v.at[pl.ds(j * _K, _K)]],
                         sem, add=True)

    def drain_scatter(r, sem):
        pltpu.make_async_copy(r, acc_sh.at[dst_v.at[pl.ds(0, _K)]],
                              sem).wait()

    gather(0, r0, g0)

    def chunk(i, carry):
        def even(_):
            @pl.when(i > 0)
            def _():
                drain_scatter(r1, ss1)
            gather(i + 1, r1, g1)
            drain_gather(r0, g0)
            scatter(i, r0, ss0)
            return 0

        def odd(_):
            drain_scatter(r0, ss0)
            gather(i + 1, r0, g0)
            drain_gather(r1, g1)
            scatter(i, r1, ss1)
            return 0

        return lax.cond(i % 2 == 0, even, odd, 0)

    lax.fori_loop(0, _NCHUNK - 1, chunk, 0)
    drain_scatter(r1, ss1)
    drain_gather(r0, g0)
    scatter(_NCHUNK - 1, r0, ss0)
    drain_scatter(r0, ss0)
    plsc.subcore_barrier()
    pltpu.sync_copy(acc_sh.at[pl.ds(rstart, _RPT)],
                    out_hbm.at[c, pl.ds(rstart, _RPT)])


@functools.partial(
    pl.kernel,
    out_type=jax.ShapeDtypeStruct((_NC, _WPAD, _H), jnp.float32),
    mesh=_sc_mesh,
    compiler_params=pltpu.CompilerParams(needs_layout_passes=False),
    scratch_types=[
        pltpu.VMEM((_EPAD,), jnp.int32),
        pltpu.VMEM((_EPAD,), jnp.int32),
        pltpu.VMEM((16,), jnp.int32),
        pltpu.VMEM((_KW, _H), jnp.float32),
        pltpu.VMEM((_KW, _H), jnp.float32),
        pltpu.VMEM_SHARED((_WPAD, _H), jnp.float32),
        pltpu.SemaphoreType.DMA,
        pltpu.SemaphoreType.DMA,
        pltpu.SemaphoreType.DMA,
        pltpu.SemaphoreType.DMA,
    ],
)
def _wagg_kernel(fsrc_hbm, fdst_hbm, fcnt_hbm, table_hbm, zeros_hbm, out_hbm,
                 src_v, dst_v, cnt_v, r0, r1, acc_sh, g0, g1, ss0, ss1):
    c = lax.axis_index("c")
    s = lax.axis_index("s")
    w = c * _NS + s
    pltpu.sync_copy(fsrc_hbm.at[pl.ds(w * _EPAD, _EPAD)], src_v)
    pltpu.sync_copy(fdst_hbm.at[pl.ds(w * _EPAD, _EPAD)], dst_v)
    pltpu.sync_copy(fcnt_hbm.at[pl.ds(w * 16, 16)], cnt_v)
    pltpu.sync_copy(zeros_hbm.at[pl.ds(0, _WRPT)],
                    acc_sh.at[pl.ds(s * _WRPT, _WRPT)])
    plsc.subcore_barrier()

    cnt = jnp.max(cnt_v[...])
    last = jnp.maximum((cnt + (_KW - 1)) // _KW, 1) - 1

    def gather(j, r, sem):
        pltpu.async_copy(table_hbm.at[src_v.at[pl.ds(j * _KW, _KW)]], r, sem)

    def drain_gather(r, sem):
        pltpu.make_async_copy(table_hbm.at[src_v.at[pl.ds(0, _KW)]], r,
                              sem).wait()

    def scatter(j, r, sem):
        pltpu.async_copy(r, acc_sh.at[dst_v.at[pl.ds(j * _KW, _KW)]],
                         sem, add=True)

    def drain_scatter(r, sem):
        pltpu.make_async_copy(r, acc_sh.at[dst_v.at[pl.ds(0, _KW)]],
                              sem).wait()

    gather(0, r0, g0)

    def chunk(i, carry):
        def even(_):
            @pl.when(i > 0)
            def _():
                drain_scatter(r1, ss1)
            gather(i + 1, r1, g1)
            drain_gather(r0, g0)
            scatter(i, r0, ss0)
            return 0

        def odd(_):
            drain_scatter(r0, ss0)
            gather(i + 1, r0, g0)
            drain_gather(r1, g1)
            scatter(i, r1, ss1)
            return 0

        return lax.cond(i % 2 == 0, even, odd, 0)

    lax.fori_loop(0, last, chunk, 0)

    def ep_even(_):
        @pl.when(last > 0)
        def _():
            drain_scatter(r1, ss1)
        drain_gather(r0, g0)
        scatter(last, r0, ss0)
        drain_scatter(r0, ss0)
        return 0

    def ep_odd(_):
        drain_scatter(r0, ss0)
        drain_gather(r1, g1)
        scatter(last, r1, ss1)
        drain_scatter(r1, ss1)
        return 0

    lax.cond(last % 2 == 0, ep_even, ep_odd, 0)
    plsc.subcore_barrier()
    pltpu.sync_copy(acc_sh.at[pl.ds(s * _WRPT, _WRPT)],
                    out_hbm.at[c, pl.ds(s * _WRPT, _WRPT)])


def _mm_body(x_ref, w_ref, o_ref):
    o_ref[...] = jnp.dot(x_ref[...], w_ref[...],
                         preferred_element_type=jnp.float32)


_mm_call = pl.pallas_call(
    _mm_body,
    grid=(_GRID_TC,),
    in_specs=[
        pl.BlockSpec((_ROWS_TC, _D), lambda i: (i, 0)),
        pl.BlockSpec((_D, _H), lambda i: (0, 0)),
    ],
    out_specs=pl.BlockSpec((_ROWS_TC, _H), lambda i: (i, 0)),
    out_shape=jax.ShapeDtypeStruct((_N, _H), jnp.float32),
)


def _dinv_of(deg_ref):
    d = deg_ref[0] + deg_ref[1] + 1.0
    return lax.rsqrt(d)


def _scale_body(deg_ref, xw_ref, o_ref):
    o_ref[...] = xw_ref[...] * _dinv_of(deg_ref)


_scale_call = pl.pallas_call(
    _scale_body,
    grid=(_GRID_TC,),
    in_specs=[
        pl.BlockSpec((_NC, _ROWS_TC, 1), lambda i: (0, i, 0)),
        pl.BlockSpec((_ROWS_TC, _H), lambda i: (i, 0)),
    ],
    out_specs=pl.BlockSpec((_ROWS_TC, _H), lambda i: (i, 0)),
    out_shape=jax.ShapeDtypeStruct((_N, _H), jnp.float32),
)


def _mid_body(deg_ref, acc_ref, xws_ref, b1_ref, w2_ref, o_ref):
    dinv = _dinv_of(deg_ref)
    h1 = acc_ref[0] + acc_ref[1] + xws_ref[...]
    h1 = jnp.maximum(h1 * dinv + b1_ref[...], 0.0)
    o_ref[...] = jnp.dot(h1, w2_ref[...],
                         preferred_element_type=jnp.float32) * dinv


_mid_call = pl.pallas_call(
    _mid_body,
    grid=(_GRID_TC,),
    in_specs=[
        pl.BlockSpec((_NC, _ROWS_TC, 1), lambda i: (0, i, 0)),
        pl.BlockSpec((_NC, _ROWS_TC, _H), lambda i: (0, i, 0)),
        pl.BlockSpec((_ROWS_TC, _H), lambda i: (i, 0)),
        pl.BlockSpec((1, _H), lambda i: (0, 0)),
        pl.BlockSpec((_H, _H), lambda i: (0, 0)),
    ],
    out_specs=pl.BlockSpec((_ROWS_TC, _H), lambda i: (i, 0)),
    out_shape=jax.ShapeDtypeStruct((_N, _H), jnp.float32),
)


def _head_body(deg_ref, acc_ref, xws_ref, b2_ref, wm1_ref, bm1_ref,
               wm2_ref, bm2_ref, o_ref):
    dinv = _dinv_of(deg_ref)
    h2 = (acc_ref[0] + acc_ref[1] + xws_ref[...]) * dinv + b2_ref[...]
    t = jnp.maximum(jnp.dot(h2, wm1_ref[...],
                            preferred_element_type=jnp.float32)
                    + bm1_ref[...], 0.0)
    o = jnp.dot(t, wm2_ref[...],
                preferred_element_type=jnp.float32) + bm2_ref[...]
    o = o - jnp.max(o, axis=1, keepdims=True)
    e = jnp.exp(o)
    o_ref[...] = e / jnp.sum(e, axis=1, keepdims=True)


_head_call = pl.pallas_call(
    _head_body,
    in_specs=[
        pl.BlockSpec((_NC, _B, 1), lambda: (0, 0, 0)),
        pl.BlockSpec((_NC, _B, _H), lambda: (0, 0, 0)),
        pl.BlockSpec((_B, _H), lambda: (0, 0)),
        pl.BlockSpec((1, _H), lambda: (0, 0)),
        pl.BlockSpec((_H, _H), lambda: (0, 0)),
        pl.BlockSpec((1, _H), lambda: (0, 0)),
        pl.BlockSpec((_H, _OUT), lambda: (0, 0)),
        pl.BlockSpec((1, _OUT), lambda: (0, 0)),
    ],
    out_specs=pl.BlockSpec((_B, _OUT), lambda: (0, 0)),
    out_shape=jax.ShapeDtypeStruct((_B, _OUT), jnp.float32),
)


def kernel(x, edge_index, n2v, batch_size,
           W1, b1, W2, b2, Wm1, bm1, Wm2, bm2):
    del n2v
    e = edge_index.astype(jnp.int32)
    src, dst = e[0], e[1]
    zerosH = jnp.zeros((_RPT, _H), jnp.float32)

    wl = jnp.clip(batch_size - _B, 0, _N - _B).astype(jnp.int32)
    win16 = jnp.broadcast_to(wl, (16,))
    deg_flat, fsrc, fdst, fcnt = _deg_kernel(src, dst, win16)
    deg3 = deg_flat.reshape(_NC, _NPAD, 1)
    xw1 = _mm_call(x, W1)
    xws1 = _scale_call(deg3, xw1)
    acc1 = _agg_kernel(src, dst, xws1, zerosH)
    xws2 = _mid_call(deg3, acc1, xws1, b1.reshape(1, _H), W2)
    acc_w = _wagg_kernel(fsrc, fdst, fcnt, xws2, zerosH)

    deg_w = lax.dynamic_slice_in_dim(deg3, wl, _B, axis=1)
    xws_w = lax.dynamic_slice_in_dim(xws2, wl, _B, axis=0)
    return _head_call(deg_w, acc_w[:, :_B], xws_w, b2.reshape(1, _H),
                      Wm1, bm1.reshape(1, _H), Wm2, bm2.reshape(1, _OUT))
